# baseline (speedup 1.0000x reference)
import functools

import jax
import jax.numpy as jnp
from jax import lax
from jax.experimental import pallas as pl
from jax.experimental.pallas import tpu as pltpu
from jax.experimental.pallas import tpu_sc as plsc

NUM_REL = 4
NC, NS, LANES = 2, 16, 16
NW = NC * NS
B = 128


def _make_scale_kernel(n_pad4, nb_cnt, nb_scl):
    mesh = plsc.VectorSubcoreMesh(core_axis_name="c", subcore_axis_name="s",
                                  num_cores=NC, num_subcores=NS)
    ztile = n_pad4 // NS

    @functools.partial(
        pl.kernel, mesh=mesh,
        out_type=jax.ShapeDtypeStruct((NW, nb_scl, B), jnp.float32),
        scratch_types=[
            pltpu.VMEM((nb_cnt, B), jnp.int32),
            pltpu.VMEM((nb_scl, B), jnp.int32),
            pltpu.VMEM((nb_scl, B), jnp.float32),
            pltpu.VMEM((B,), jnp.float32),
            pltpu.VMEM((B,), jnp.float32),
            pltpu.VMEM((ztile,), jnp.float32),
            pltpu.VMEM_SHARED((n_pad4,), jnp.float32),
            pltpu.SemaphoreType.DMA,
        ],
    )
    def scale_kernel(cidx_scl, scale_out,
                     ci_v, cs_v, sc_v, cb_v, ones_v, z_v, ctab, sem):
        cid = lax.axis_index("c")
        sid = lax.axis_index("s")
        wid = cid * NS + sid

        one16 = jnp.ones((LANES,), jnp.float32)
        zero16 = jnp.zeros((LANES,), jnp.float32)
        for j in range(B // LANES):
            ones_v[pl.ds(j * LANES, LANES)] = one16

        def zfill(i, _):
            z_v[pl.ds(i * LANES, LANES)] = zero16
            return 0
        lax.fori_loop(0, ztile // LANES, zfill, 0)
        pltpu.sync_copy(z_v, ctab.at[pl.ds(sid * ztile, ztile)])
        plsc.subcore_barrier()

        pltpu.sync_copy(cidx_scl.at[2 * sid], ci_v.at[pl.ds(0, nb_cnt // 2)])
        pltpu.sync_copy(cidx_scl.at[2 * sid + 1],
                        ci_v.at[pl.ds(nb_cnt // 2, nb_cnt // 2)])

        def fire(b, _):
            pltpu.async_copy(ones_v, ctab.at[ci_v.at[b]], sem, add=True)
            return 0
        lax.fori_loop(0, nb_cnt, fire, 0)

        def drain(b, _):
            pltpu.make_async_copy(ones_v, ctab.at[ci_v.at[b]], sem).wait()
            return 0
        lax.fori_loop(0, nb_cnt, drain, 0)
        plsc.subcore_barrier()

        pltpu.sync_copy(cidx_scl.at[wid], cs_v)

        def sbody(b, _):
            pltpu.sync_copy(ctab.at[cs_v.at[b]], cb_v)
            for j in range(B // LANES):
                sl = pl.ds(j * LANES, LANES)
                sc_v[b, sl] = 1.0 / jnp.maximum(cb_v[sl], 1.0)
            return 0
        lax.fori_loop(0, nb_scl, sbody, 0)
        pltpu.sync_copy(sc_v, scale_out.at[wid])

    return scale_kernel


def _make_agg_kernel(n_tab, n_acc, kblk, d, ba, nbuf):
    mesh = plsc.VectorSubcoreMesh(core_axis_name="c", subcore_axis_name="s",
                                  num_cores=NC, num_subcores=NS)
    ztile = n_acc // NS
    k0, k1 = kblk

    @functools.partial(
        pl.kernel, mesh=mesh,
        out_type=jax.ShapeDtypeStruct((NC, n_acc, d), jnp.float32),
        scratch_types=(
            [pltpu.VMEM((ba,), jnp.int32) for _ in range(nbuf)]
            + [pltpu.VMEM((nbuf, ba), jnp.int32)]
            + [pltpu.VMEM((ba,), jnp.float32) for _ in range(nbuf)]
            + [pltpu.VMEM((ba, d), jnp.float32) for _ in range(nbuf)]
            + [pltpu.VMEM_SHARED((n_acc, d), jnp.float32)]
            + [pltpu.SemaphoreType.DMA] * (4 * nbuf)
        ),
    )
    def agg_kernel(tab, g3, d3, s3, znd, parts, *bufs_and_sems):
        gbufs = bufs_and_sems[:nbuf]
        db = bufs_and_sems[nbuf]
        sbufs = bufs_and_sems[nbuf + 1:2 * nbuf + 1]
        bufs = bufs_and_sems[2 * nbuf + 1:3 * nbuf + 1]
        acc = bufs_and_sems[3 * nbuf + 1]
        sems = bufs_and_sems[3 * nbuf + 2:]
        egs, eds = sems[:nbuf], sems[nbuf:2 * nbuf]
        gsems, ssems = sems[2 * nbuf:3 * nbuf], sems[3 * nbuf:4 * nbuf]

        cid = lax.axis_index("c")
        sid = lax.axis_index("s")
        base = jnp.where(cid == 0, sid * k0, NS * k0 + sid * k1)
        nblk = jnp.where(cid == 0, k0, k1)

        pltpu.sync_copy(znd.at[pl.ds(sid * ztile, ztile)],
                        acc.at[pl.ds(sid * ztile, ztile)])
        plsc.subcore_barrier()

        def scale_rows(buf, sb):
            def gbody(g, _):
                s16 = sb[pl.ds(g * LANES, LANES)]
                for el in range(LANES):
                    s = s16[el]
                    e = g * LANES + el
                    for j in range(d // LANES):
                        sl = pl.ds(j * LANES, LANES)
                        buf[e, sl] = buf[e, sl] * s
                return 0
            lax.fori_loop(0, ba // LANES, gbody, 0)

        pltpu.sync_copy(g3.at[base], gbufs[0])
        pltpu.async_copy(tab.at[gbufs[0]], bufs[0], gsems[0])
        for j in range(1, nbuf - 1):
            pltpu.async_copy(g3.at[base + j], gbufs[j], egs[j])
        for j in range(nbuf - 1):
            pltpu.async_copy(d3.at[base + j], db.at[j], eds[j])
            pltpu.async_copy(s3.at[base + j], sbufs[j], eds[j])
        for j in range(1, nbuf - 1):
            pltpu.make_async_copy(g3.at[base], gbufs[j], egs[j]).wait()
            pltpu.async_copy(tab.at[gbufs[j]], bufs[j], gsems[j])
        pltpu.async_copy(g3.at[base + nbuf - 1], gbufs[nbuf - 1],
                         egs[nbuf - 1])

        def body(k2, _):
            for ph in range(nbuf):
                k = nbuf * k2 + ph
                phm1 = (ph - 1) % nbuf
                @pl.when(k > 0)
                def _():
                    pltpu.make_async_copy(bufs[phm1], acc.at[db.at[phm1]],
                                          ssems[phm1]).wait()
                @pl.when(k + nbuf - 1 < nblk)
                def _():
                    pltpu.make_async_copy(g3.at[base], gbufs[phm1],
                                          egs[phm1]).wait()
                    pltpu.async_copy(tab.at[gbufs[phm1]], bufs[phm1],
                                     gsems[phm1])
                    pltpu.async_copy(d3.at[base + k + nbuf - 1], db.at[phm1],
                                     eds[phm1])
                    pltpu.async_copy(s3.at[base + k + nbuf - 1], sbufs[phm1],
                                     eds[phm1])
                pltpu.make_async_copy(tab.at[gbufs[ph]], bufs[ph],
                                      gsems[ph]).wait()
                @pl.when(k + nbuf < nblk)
                def _():
                    pltpu.async_copy(g3.at[base + k + nbuf], gbufs[ph],
                                     egs[ph])
                pltpu.make_async_copy(d3.at[base], db.at[ph],
                                      eds[ph]).wait()
                pltpu.make_async_copy(s3.at[base], sbufs[ph],
                                      eds[ph]).wait()
                scale_rows(bufs[ph], sbufs[ph])
                pltpu.async_copy(bufs[ph], acc.at[db.at[ph]], ssems[ph],
                                 add=True)
            return 0
        lax.fori_loop(0, nblk // nbuf, body, 0)

        pltpu.make_async_copy(bufs[nbuf - 1], acc.at[db.at[nbuf - 1]],
                              ssems[nbuf - 1]).wait()

        plsc.subcore_barrier()
        pltpu.sync_copy(acc.at[pl.ds(sid * ztile, ztile)],
                        parts.at[cid, pl.ds(sid * ztile, ztile)])

    return agg_kernel


def _mm5_body(x_ref, w_ref, b_ref, o_ref):
    r = pl.program_id(0)
    y = jnp.dot(x_ref[...], w_ref[0], preferred_element_type=jnp.float32)
    o_ref[0] = y + jnp.where(r == NUM_REL, 1.0, 0.0) * b_ref[...]


def _mm5_fused_body(base_ref, p_ref, w_ref, b_ref, o_ref):
    r = pl.program_id(0)
    h = jnp.maximum(base_ref[...] + p_ref[0] + p_ref[1], 0.0)
    y = jnp.dot(h, w_ref[0], preferred_element_type=jnp.float32)
    o_ref[0] = y + jnp.where(r == NUM_REL, 1.0, 0.0) * b_ref[...]


def _combine_body(base_ref, p_ref, o_ref):
    o_ref[...] = base_ref[...] + p_ref[0] + p_ref[1]


def _tc_mm5(x, wcat, bias, bn):
    n, d = x.shape
    grid = (NUM_REL + 1, n // bn)
    return pl.pallas_call(
        _mm5_body,
        grid=grid,
        in_specs=[
            pl.BlockSpec((bn, d), lambda r, i: (i, 0)),
            pl.BlockSpec((1, d, d), lambda r, i: (r, 0, 0)),
            pl.BlockSpec((d,), lambda r, i: (0,)),
        ],
        out_specs=pl.BlockSpec((1, bn, d), lambda r, i: (r, i, 0)),
        out_shape=jax.ShapeDtypeStruct((NUM_REL + 1, n, d), jnp.float32),
    )(x, wcat, bias)


def _tc_mm5_fused(base, parts, wcat, bias, bn):
    n, d = base.shape
    grid = (NUM_REL + 1, n // bn)
    return pl.pallas_call(
        _mm5_fused_body,
        grid=grid,
        in_specs=[
            pl.BlockSpec((bn, d), lambda r, i: (i, 0)),
            pl.BlockSpec((NC, bn, d), lambda r, i: (0, i, 0)),
            pl.BlockSpec((1, d, d), lambda r, i: (r, 0, 0)),
            pl.BlockSpec((d,), lambda r, i: (0,)),
        ],
        out_specs=pl.BlockSpec((1, bn, d), lambda r, i: (r, i, 0)),
        out_shape=jax.ShapeDtypeStruct((NUM_REL + 1, n, d), jnp.float32),
    )(base, parts, wcat, bias)


def _tc_combine(base, parts, bn):
    n, d = base.shape
    return pl.pallas_call(
        _combine_body,
        grid=(n // bn,),
        in_specs=[
            pl.BlockSpec((bn, d), lambda i: (i, 0)),
            pl.BlockSpec((NC, bn, d), lambda i: (0, i, 0)),
        ],
        out_specs=pl.BlockSpec((bn, d), lambda i: (i, 0)),
        out_shape=jax.ShapeDtypeStruct((n, d), jnp.float32),
    )(base, parts)


def kernel(edge_index, x_init, edge_type, weight1, root1, bias1,
           weight2, root2, bias2):
    n, d = x_init.shape
    e = edge_index.shape[1]
    src, dst = edge_index[0], edge_index[1]
    et = edge_type

    nb = -(-e // (NW * B))
    if nb % 2:
        nb += 1
    e_pad = NW * nb * B
    nb_cnt = e_pad // (NS * B)
    pad = e_pad - e

    n_acc = -(-(n + 1) // (NS * 8)) * (NS * 8)
    n_tab = NUM_REL * n
    n_pad4 = -(-(n_tab + 1) // (NS * B)) * (NS * B)

    i32 = jnp.int32
    totb = e_pad // B
    k0, k1 = 42, 118
    assert NS * (k0 + k1) == totb
    g = (et * n + src).astype(i32)
    cidx = (et * n + dst).astype(i32)
    g_p = jnp.concatenate([g, jnp.zeros((pad,), i32)]).reshape(totb, B)
    dst_p = jnp.concatenate([dst.astype(i32),
                             jnp.full((pad,), n, i32)]).reshape(totb, B)
    cidx_p = jnp.concatenate([cidx, jnp.full((pad,), n_tab, i32)])
    cidx_scl = cidx_p.reshape(NW, nb, B)

    scale3 = _make_scale_kernel(n_pad4, nb_cnt, nb)(cidx_scl)
    scale3 = scale3.reshape(totb, B)

    znd = jnp.zeros((n_acc, d), jnp.float32)
    agg = _make_agg_kernel(n_tab, n_acc, (k0, k1), d, B, 2)

    bn = 2000
    wcat1 = jnp.concatenate([weight1, root1[None]], axis=0)
    wcat2 = jnp.concatenate([weight2, root2[None]], axis=0)

    y1 = _tc_mm5(x_init, wcat1, bias1, bn)
    tab1 = y1[:NUM_REL].reshape(n_tab, d)
    base1 = y1[NUM_REL]
    parts1 = agg(tab1, g_p, dst_p, scale3, znd)[:, :n, :]

    y2 = _tc_mm5_fused(base1, parts1, wcat2, bias2, bn)
    tab2 = y2[:NUM_REL].reshape(n_tab, d)
    base2 = y2[NUM_REL]
    parts2 = agg(tab2, g_p, dst_p, scale3, znd)[:, :n, :]

    return _tc_combine(base2, parts2, bn)


# speedup vs baseline: 1.0675x; 1.0675x over previous
import functools

import jax
import jax.numpy as jnp
from jax import lax
from jax.experimental import pallas as pl
from jax.experimental.pallas import tpu as pltpu
from jax.experimental.pallas import tpu_sc as plsc

NUM_REL = 4
NC, NS, LANES = 2, 16, 16
NW = NC * NS
B = 128


def _make_scale_kernel(n_pad4, nb_cnt, nb_scl):
    mesh = plsc.VectorSubcoreMesh(core_axis_name="c", subcore_axis_name="s",
                                  num_cores=NC, num_subcores=NS)
    ztile = n_pad4 // NS

    @functools.partial(
        pl.kernel, mesh=mesh,
        out_type=jax.ShapeDtypeStruct((NW, nb_scl, B), jnp.float32),
        scratch_types=[
            pltpu.VMEM((nb_cnt, B), jnp.int32),
            pltpu.VMEM((nb_scl, B), jnp.int32),
            pltpu.VMEM((nb_scl, B), jnp.float32),
            pltpu.VMEM((B,), jnp.float32),
            pltpu.VMEM((B,), jnp.float32),
            pltpu.VMEM((ztile,), jnp.float32),
            pltpu.VMEM_SHARED((n_pad4,), jnp.float32),
            pltpu.SemaphoreType.DMA,
        ],
    )
    def scale_kernel(cidx_scl, scale_out,
                     ci_v, cs_v, sc_v, cb_v, ones_v, z_v, ctab, sem):
        cid = lax.axis_index("c")
        sid = lax.axis_index("s")
        wid = cid * NS + sid

        one16 = jnp.ones((LANES,), jnp.float32)
        zero16 = jnp.zeros((LANES,), jnp.float32)
        for j in range(B // LANES):
            ones_v[pl.ds(j * LANES, LANES)] = one16

        def zfill(i, _):
            z_v[pl.ds(i * LANES, LANES)] = zero16
            return 0
        lax.fori_loop(0, ztile // LANES, zfill, 0)
        pltpu.sync_copy(z_v, ctab.at[pl.ds(sid * ztile, ztile)])
        plsc.subcore_barrier()

        pltpu.sync_copy(cidx_scl.at[2 * sid], ci_v.at[pl.ds(0, nb_cnt // 2)])
        pltpu.sync_copy(cidx_scl.at[2 * sid + 1],
                        ci_v.at[pl.ds(nb_cnt // 2, nb_cnt // 2)])

        def fire(b, _):
            pltpu.async_copy(ones_v, ctab.at[ci_v.at[b]], sem, add=True)
            return 0
        lax.fori_loop(0, nb_cnt, fire, 0)

        def drain(b, _):
            pltpu.make_async_copy(ones_v, ctab.at[ci_v.at[b]], sem).wait()
            return 0
        lax.fori_loop(0, nb_cnt, drain, 0)
        plsc.subcore_barrier()

        pltpu.sync_copy(cidx_scl.at[wid], cs_v)

        def sbody(b, _):
            pltpu.sync_copy(ctab.at[cs_v.at[b]], cb_v)
            for j in range(B // LANES):
                sl = pl.ds(j * LANES, LANES)
                sc_v[b, sl] = 1.0 / jnp.maximum(cb_v[sl], 1.0)
            return 0
        lax.fori_loop(0, nb_scl, sbody, 0)
        pltpu.sync_copy(sc_v, scale_out.at[wid])

    return scale_kernel


def _make_agg_kernel(n_tab, n_acc, kblk, d, ba, nbuf):
    mesh = plsc.VectorSubcoreMesh(core_axis_name="c", subcore_axis_name="s",
                                  num_cores=NC, num_subcores=NS)
    ztile = n_acc // NS
    k0, k1 = kblk

    @functools.partial(
        pl.kernel, mesh=mesh,
        out_type=jax.ShapeDtypeStruct((NC, n_acc, d), jnp.float32),
        scratch_types=(
            [pltpu.VMEM((ba,), jnp.int32) for _ in range(nbuf)]
            + [pltpu.VMEM((nbuf, ba), jnp.int32)]
            + [pltpu.VMEM((ba,), jnp.float32) for _ in range(nbuf)]
            + [pltpu.VMEM((ba, d), jnp.float32) for _ in range(nbuf)]
            + [pltpu.VMEM_SHARED((n_acc, d), jnp.float32)]
            + [pltpu.SemaphoreType.DMA] * (4 * nbuf)
        ),
    )
    def agg_kernel(tab, g3, d3, s3, znd, parts, *bufs_and_sems):
        gbufs = bufs_and_sems[:nbuf]
        db = bufs_and_sems[nbuf]
        sbufs = bufs_and_sems[nbuf + 1:2 * nbuf + 1]
        bufs = bufs_and_sems[2 * nbuf + 1:3 * nbuf + 1]
        acc = bufs_and_sems[3 * nbuf + 1]
        sems = bufs_and_sems[3 * nbuf + 2:]
        egs, eds = sems[:nbuf], sems[nbuf:2 * nbuf]
        gsems, ssems = sems[2 * nbuf:3 * nbuf], sems[3 * nbuf:4 * nbuf]

        cid = lax.axis_index("c")
        sid = lax.axis_index("s")
        base = jnp.where(cid == 0, sid * k0, NS * k0 + sid * k1)
        nblk = jnp.where(cid == 0, k0, k1)

        pltpu.sync_copy(znd.at[pl.ds(sid * ztile, ztile)],
                        acc.at[pl.ds(sid * ztile, ztile)])
        plsc.subcore_barrier()

        def scale_rows(buf, sb):
            def gbody(g, _):
                s16 = sb[pl.ds(g * LANES, LANES)]
                for el in range(LANES):
                    s = s16[el]
                    e = g * LANES + el
                    for j in range(d // LANES):
                        sl = pl.ds(j * LANES, LANES)
                        buf[e, sl] = buf[e, sl] * s
                return 0
            lax.fori_loop(0, ba // LANES, gbody, 0)

        pltpu.sync_copy(g3.at[base], gbufs[0])
        pltpu.async_copy(tab.at[gbufs[0]], bufs[0], gsems[0])
        for j in range(1, nbuf - 1):
            pltpu.async_copy(g3.at[base + j], gbufs[j], egs[j])
        for j in range(nbuf - 1):
            pltpu.async_copy(d3.at[base + j], db.at[j], eds[j])
            pltpu.async_copy(s3.at[base + j], sbufs[j], eds[j])
        for j in range(1, nbuf - 1):
            pltpu.make_async_copy(g3.at[base], gbufs[j], egs[j]).wait()
            pltpu.async_copy(tab.at[gbufs[j]], bufs[j], gsems[j])
        pltpu.async_copy(g3.at[base + nbuf - 1], gbufs[nbuf - 1],
                         egs[nbuf - 1])

        def body(k2, _):
            for ph in range(nbuf):
                k = nbuf * k2 + ph
                phm1 = (ph - 1) % nbuf
                @pl.when(k > 0)
                def _():
                    pltpu.make_async_copy(bufs[phm1], acc.at[db.at[phm1]],
                                          ssems[phm1]).wait()
                @pl.when(k + nbuf - 1 < nblk)
                def _():
                    pltpu.make_async_copy(g3.at[base], gbufs[phm1],
                                          egs[phm1]).wait()
                    pltpu.async_copy(tab.at[gbufs[phm1]], bufs[phm1],
                                     gsems[phm1])
                    pltpu.async_copy(d3.at[base + k + nbuf - 1], db.at[phm1],
                                     eds[phm1])
                    pltpu.async_copy(s3.at[base + k + nbuf - 1], sbufs[phm1],
                                     eds[phm1])
                pltpu.make_async_copy(tab.at[gbufs[ph]], bufs[ph],
                                      gsems[ph]).wait()
                @pl.when(k + nbuf < nblk)
                def _():
                    pltpu.async_copy(g3.at[base + k + nbuf], gbufs[ph],
                                     egs[ph])
                pltpu.make_async_copy(d3.at[base], db.at[ph],
                                      eds[ph]).wait()
                pltpu.make_async_copy(s3.at[base], sbufs[ph],
                                      eds[ph]).wait()
                scale_rows(bufs[ph], sbufs[ph])
                pltpu.async_copy(bufs[ph], acc.at[db.at[ph]], ssems[ph],
                                 add=True)
            return 0
        lax.fori_loop(0, nblk // nbuf, body, 0)

        pltpu.make_async_copy(bufs[nbuf - 1], acc.at[db.at[nbuf - 1]],
                              ssems[nbuf - 1]).wait()

        plsc.subcore_barrier()
        pltpu.sync_copy(acc.at[pl.ds(sid * ztile, ztile)],
                        parts.at[cid, pl.ds(sid * ztile, ztile)])

    return agg_kernel


def _mm5_body(x_ref, w_ref, b_ref, o_ref):
    r = pl.program_id(0)
    y = jnp.dot(x_ref[...], w_ref[0], preferred_element_type=jnp.float32)
    o_ref[0] = y + jnp.where(r == NUM_REL, 1.0, 0.0) * b_ref[...]


def _mm5_fused_body(base_ref, p_ref, w_ref, b_ref, o_ref):
    r = pl.program_id(0)
    h = jnp.maximum(base_ref[...] + p_ref[0] + p_ref[1], 0.0)
    y = jnp.dot(h, w_ref[0], preferred_element_type=jnp.float32)
    o_ref[0] = y + jnp.where(r == NUM_REL, 1.0, 0.0) * b_ref[...]


def _combine_body(base_ref, p_ref, o_ref):
    o_ref[...] = base_ref[...] + p_ref[0] + p_ref[1]


def _tc_mm5(x, wcat, bias, bn):
    n, d = x.shape
    grid = (NUM_REL + 1, n // bn)
    return pl.pallas_call(
        _mm5_body,
        grid=grid,
        in_specs=[
            pl.BlockSpec((bn, d), lambda r, i: (i, 0)),
            pl.BlockSpec((1, d, d), lambda r, i: (r, 0, 0)),
            pl.BlockSpec((d,), lambda r, i: (0,)),
        ],
        out_specs=pl.BlockSpec((1, bn, d), lambda r, i: (r, i, 0)),
        out_shape=jax.ShapeDtypeStruct((NUM_REL + 1, n, d), jnp.float32),
    )(x, wcat, bias)


def _tc_mm5_fused(base, parts, wcat, bias, bn):
    n, d = base.shape
    grid = (NUM_REL + 1, n // bn)
    return pl.pallas_call(
        _mm5_fused_body,
        grid=grid,
        in_specs=[
            pl.BlockSpec((bn, d), lambda r, i: (i, 0)),
            pl.BlockSpec((NC, bn, d), lambda r, i: (0, i, 0)),
            pl.BlockSpec((1, d, d), lambda r, i: (r, 0, 0)),
            pl.BlockSpec((d,), lambda r, i: (0,)),
        ],
        out_specs=pl.BlockSpec((1, bn, d), lambda r, i: (r, i, 0)),
        out_shape=jax.ShapeDtypeStruct((NUM_REL + 1, n, d), jnp.float32),
    )(base, parts, wcat, bias)


def _tc_combine(base, parts, bn):
    n, d = base.shape
    return pl.pallas_call(
        _combine_body,
        grid=(n // bn,),
        in_specs=[
            pl.BlockSpec((bn, d), lambda i: (i, 0)),
            pl.BlockSpec((NC, bn, d), lambda i: (0, i, 0)),
        ],
        out_specs=pl.BlockSpec((bn, d), lambda i: (i, 0)),
        out_shape=jax.ShapeDtypeStruct((n, d), jnp.float32),
    )(base, parts)


def kernel(edge_index, x_init, edge_type, weight1, root1, bias1,
           weight2, root2, bias2):
    n, d = x_init.shape
    e = edge_index.shape[1]
    src, dst = edge_index[0], edge_index[1]
    et = edge_type

    nb = -(-e // (NW * B))
    if nb % 2:
        nb += 1
    e_pad = NW * nb * B
    nb_cnt = e_pad // (NS * B)
    pad = e_pad - e

    n_acc = -(-(n + 1) // (NS * 8)) * (NS * 8)
    n_tab = NUM_REL * n
    n_pad4 = -(-(n_tab + 1) // (NS * B)) * (NS * B)

    i32 = jnp.int32
    totb = e_pad // B
    k0, k1 = nb, nb
    assert NS * (k0 + k1) == totb
    g = (et * n + src).astype(i32)
    cidx = (et * n + dst).astype(i32)
    g_p = jnp.concatenate([g, jnp.zeros((pad,), i32)]).reshape(totb, B)
    dst_p = jnp.concatenate([dst.astype(i32),
                             jnp.full((pad,), n, i32)]).reshape(totb, B)
    cidx_p = jnp.concatenate([cidx, jnp.full((pad,), n_tab, i32)])
    cidx_scl = cidx_p.reshape(NW, nb, B)

    scale3 = _make_scale_kernel(n_pad4, nb_cnt, nb)(cidx_scl)
    scale3 = scale3.reshape(totb, B)

    znd = jnp.zeros((n_acc, d), jnp.float32)
    agg = _make_agg_kernel(n_tab, n_acc, (k0, k1), d, B, 2)

    bn = 2000
    wcat1 = jnp.concatenate([weight1, root1[None]], axis=0)
    wcat2 = jnp.concatenate([weight2, root2[None]], axis=0)

    y1 = _tc_mm5(x_init, wcat1, bias1, bn)
    tab1 = y1[:NUM_REL].reshape(n_tab, d)
    base1 = y1[NUM_REL]
    parts1 = agg(tab1, g_p, dst_p, scale3, znd)[:, :n, :]

    y2 = _tc_mm5_fused(base1, parts1, wcat2, bias2, bn)
    tab2 = y2[:NUM_REL].reshape(n_tab, d)
    base2 = y2[NUM_REL]
    parts2 = agg(tab2, g_p, dst_p, scale3, znd)[:, :n, :]

    return _tc_combine(base2, parts2, bn)
